# Initial kernel scaffold; baseline (speedup 1.0000x reference)
import jax
import jax.numpy as jnp
from jax.experimental import pallas as pl
from jax.experimental.pallas import tpu as pltpu

N = 10000
E = 160000
HEADS = 4
B = 16
NUM_SAT = 40


def _gat_layer(x, src, dst, W, a_s, a_d, b, heads, out_ch):
    Nn = x.shape[0]
    h = (x @ W).reshape(Nn, heads, out_ch)
    alpha_src = (h * a_s[None, :, :]).sum(-1)
    alpha_dst = (h * a_d[None, :, :]).sum(-1)
    e = jax.nn.leaky_relu(alpha_src[src] + alpha_dst[dst], negative_slope=0.2)
    emax = jax.ops.segment_max(e, dst, num_segments=Nn)
    emax = jnp.where(jnp.isfinite(emax), emax, 0.0)
    ee = jnp.exp(e - emax[dst])
    denom = jax.ops.segment_sum(ee, dst, num_segments=Nn)
    alpha = ee / (denom[dst] + 1e-16)
    out = jax.ops.segment_sum(h[src] * alpha[:, :, None], dst, num_segments=Nn)
    return out.reshape(Nn, heads * out_ch) + b


def _heads_kernel(g_ref, A1, bA1, A2, bA2, A3, bA3, C1, bC1, C2, bC2, C3, bC3,
                  logits_ref, value_ref):
    g = g_ref[...]
    a = jnp.maximum(g @ A1[...] + bA1[...][None, :], 0.0)
    a = jnp.maximum(a @ A2[...] + bA2[...][None, :], 0.0)
    logits_ref[...] = jnp.tanh(a @ A3[...] + bA3[...][None, :])
    c = jnp.maximum(g @ C1[...] + bC1[...][None, :], 0.0)
    c = jnp.maximum(c @ C2[...] + bC2[...][None, :], 0.0)
    value_ref[...] = c @ C3[...] + bC3[...][None, :]


def kernel(x, params, edge_index, batch):
    p = params
    loop = jnp.arange(N, dtype=edge_index.dtype)
    src = jnp.concatenate([edge_index[0], loop])
    dst = jnp.concatenate([edge_index[1], loop])
    h = jax.nn.relu(_gat_layer(x, src, dst, p['W1'], p['a1s'], p['a1d'], p['b1'], HEADS, 64))
    h = jax.nn.relu(_gat_layer(h, src, dst, p['W2'], p['a2s'], p['a2d'], p['b2'], HEADS, 128))
    h = jax.nn.relu(_gat_layer(h, src, dst, p['W3'], p['a3s'], p['a3d'], p['b3'], HEADS, 256))
    ones = jnp.ones((h.shape[0],), jnp.float32)
    cnt = jax.ops.segment_sum(ones, batch, num_segments=B)
    g = jax.ops.segment_sum(h, batch, num_segments=B) / jnp.maximum(cnt, 1.0)[:, None]

    logits, value = pl.pallas_call(
        _heads_kernel,
        out_shape=(
            jax.ShapeDtypeStruct((B, NUM_SAT * 2), jnp.float32),
            jax.ShapeDtypeStruct((B, 1), jnp.float32),
        ),
    )(g, p['A1'], p['bA1'], p['A2'], p['bA2'], p['A3'], p['bA3'],
      p['C1'], p['bC1'], p['C2'], p['bC2'], p['C3'], p['bC3'])
    return (logits.reshape(-1, NUM_SAT, 2), value)


# speedup vs baseline: 4.6311x; 4.6311x over previous
import functools

import jax
import jax.numpy as jnp
from jax import lax
from jax.experimental import pallas as pl
from jax.experimental.pallas import tpu as pltpu
from jax.experimental.pallas import tpu_sc as plsc

N = 10000
E = 160000
HEADS = 4
B = 16
NUM_SAT = 40

NW = 32
NPW = 320
EP = 170240
RP_PAD = 10256
ADC_PAD = NW * NPW * HEADS
ILANE = lambda: lax.broadcasted_iota(jnp.int32, (16,), 0)


def _dense_body(z_ref, b_ref, w_ref, as_ref, ad_ref, h_ref, asc_ref, adc_ref,
                *, do_relu):
    act = z_ref[...] + b_ref[...]
    if do_relu:
        act = jnp.maximum(act, 0.0)
    h = jnp.dot(act, w_ref[...], preferred_element_type=jnp.float32)
    h_ref[...] = h
    ch = w_ref.shape[1] // HEADS
    asc_cols, adc_cols = [], []
    for hd in range(HEADS):
        seg = h[:, hd * ch:(hd + 1) * ch]
        asc_cols.append(jnp.sum(seg * as_ref[hd:hd + 1, :], axis=1,
                                keepdims=True))
        adc_cols.append(jnp.sum(seg * ad_ref[hd:hd + 1, :], axis=1,
                                keepdims=True))
    asc_ref[...] = jnp.concatenate(asc_cols, axis=1)
    adc_ref[...] = jnp.concatenate(adc_cols, axis=1)


def _dense(z, bprev, W, a_s, a_d, do_relu):
    Din = z.shape[1]
    Dout = W.shape[1]
    ch = Dout // HEADS
    R = 1000
    grid = N // R
    return pl.pallas_call(
        functools.partial(_dense_body, do_relu=do_relu),
        grid=(grid,),
        in_specs=[
            pl.BlockSpec((R, Din), lambda i: (i, 0)),
            pl.BlockSpec((1, Din), lambda i: (0, 0)),
            pl.BlockSpec((Din, Dout), lambda i: (0, 0)),
            pl.BlockSpec((HEADS, ch), lambda i: (0, 0)),
            pl.BlockSpec((HEADS, ch), lambda i: (0, 0)),
        ],
        out_specs=[
            pl.BlockSpec((R, Dout), lambda i: (i, 0)),
            pl.BlockSpec((R, HEADS), lambda i: (i, 0)),
            pl.BlockSpec((R, HEADS), lambda i: (i, 0)),
        ],
        out_shape=[
            jax.ShapeDtypeStruct((N, Dout), jnp.float32),
            jax.ShapeDtypeStruct((N, HEADS), jnp.float32),
            jax.ShapeDtypeStruct((N, HEADS), jnp.float32),
        ],
    )(z, bprev.reshape(1, Din), W, a_s, a_d)


def _rd_scalar(ref, j):
    base = j & -16
    v = ref[pl.ds(base, 16)]
    m = ILANE() == (j - base)
    return jax.lax.reduce_sum(jnp.where(m, v, 0), axes=(0,))


def _make_sc_agg(D):
    CH = D // HEADS
    TG = D // 16
    CPH = CH // 16
    mesh = plsc.VectorSubcoreMesh(core_axis_name="c", subcore_axis_name="s")

    def body(h_hbm, asrc_hbm, adc_hbm, rptr_hbm, ssrc_hbm, m8_hbm, out_hbm,
             asrc_v, adc_v, rptr_v, m8_v, src24_v, idx16_v, eebuf_v, rows_v,
             out_v, sem):
        wid = lax.axis_index("s") * 2 + lax.axis_index("c")
        n0 = wid * NPW
        nn = jnp.minimum(NPW, N - n0)

        pltpu.sync_copy(asrc_hbm, asrc_v)
        pltpu.sync_copy(adc_hbm.at[pl.ds(n0 * HEADS, NPW * HEADS)], adc_v)
        pltpu.sync_copy(rptr_hbm.at[pl.ds(n0, 336)], rptr_v)
        pltpu.sync_copy(m8_hbm, m8_v)

        m16 = m8_v[pl.ds(0, 16)]
        msplat = [jnp.full((16,), jax.lax.reduce_sum(
            jnp.where(ILANE() == h, m16, 0.0), axes=(0,)), jnp.float32)
            for h in range(HEADS)]

        def edge_group(g, p0, count, adsplat, consume):
            base_e = p0 + g * 16
            ba = pl.multiple_of(base_e & -8, 8)
            off = base_e - ba
            pltpu.sync_copy(ssrc_hbm.at[pl.ds(ba, 128)], src24_v)
            sv = plsc.load_gather(src24_v, [jnp.full((16,), off, jnp.int32)
                                            + ILANE()])
            valid = (g * 16 + ILANE()) < count
            ees = []
            for h in range(HEADS):
                a = plsc.load_gather(asrc_v, [sv * HEADS + h])
                e = a + adsplat[h]
                e = jnp.where(e > 0, e, 0.2 * e)
                ee = jnp.exp(e - msplat[h])
                ees.append(jnp.where(valid, ee, 0.0))
            return consume(sv, ees)

        def node_body(nl, carry):
            p0 = _rd_scalar(rptr_v, nl)
            p1 = _rd_scalar(rptr_v, nl + 1)
            count = p1 - p0
            ngroups = (count + 15) >> 4
            adsplat = [plsc.load_gather(
                adc_v, [jnp.full((16,), nl * HEADS + h, jnp.int32)])
                for h in range(HEADS)]

            def p1_body(g, dacc):
                def consume(sv, ees):
                    return tuple(dacc[h] + ees[h] for h in range(HEADS))
                return edge_group(g, p0, count, adsplat, consume)

            dacc = lax.fori_loop(
                0, ngroups, p1_body,
                tuple(jnp.zeros((16,), jnp.float32) for _ in range(HEADS)))
            rsplat = [1.0 / jnp.full(
                (16,), lax.reduce_sum(dacc[h], axes=(0,)), jnp.float32)
                for h in range(HEADS)]

            for t in range(TG):
                out_v[pl.ds(t * 16, 16)] = jnp.zeros((16,), jnp.float32)

            def p2_body(g, c2):
                def consume(sv, ees):
                    for h in range(HEADS):
                        eebuf_v[pl.ds(h * 16, 16)] = ees[h] * rsplat[h]
                    pltpu.async_copy(h_hbm.at[sv], rows_v, sem).wait()

                    def per_edge(ei, c3):
                        esplat = jnp.full((16,), ei, jnp.int32)
                        alph = [plsc.load_gather(
                            eebuf_v, [jnp.full((16,), h * 16, jnp.int32)
                                      + esplat])
                            for h in range(HEADS)]
                        for t in range(TG):
                            r = plsc.load_gather(
                                rows_v, [esplat, t * 16 + ILANE()])
                            out_v[pl.ds(t * 16, 16)] = (
                                out_v[pl.ds(t * 16, 16)] + alph[t // CPH] * r)
                        return c3

                    lax.fori_loop(0, 16, per_edge, 0)
                    return c2
                return edge_group(g, p0, count, adsplat, consume)

            lax.fori_loop(0, ngroups, p2_body, 0)
            pltpu.sync_copy(
                out_v, out_hbm.at[pl.ds(pl.multiple_of((n0 + nl) * D, D), D)])
            return carry

        lax.fori_loop(0, nn, node_body, 0)

    return pl.kernel(
        body,
        out_type=jax.ShapeDtypeStruct((N * D,), jnp.float32),
        mesh=mesh,
        compiler_params=pltpu.CompilerParams(needs_layout_passes=False),
        scratch_types=[
            pltpu.VMEM((N * HEADS,), jnp.float32),
            pltpu.VMEM((NPW * HEADS,), jnp.float32),
            pltpu.VMEM((336,), jnp.int32),
            pltpu.VMEM((128,), jnp.float32),
            pltpu.VMEM((128,), jnp.int32),
            pltpu.VMEM((16,), jnp.int32),
            pltpu.VMEM((128,), jnp.float32),
            pltpu.VMEM((16, D), jnp.float32),
            pltpu.VMEM((D,), jnp.float32),
            pltpu.SemaphoreType.DMA,
        ],
    )


_SC_AGG = {}


def _sc_agg(D, h, asrc_flat, adc_flat, rptr, ssrc, m8):
    if D not in _SC_AGG:
        _SC_AGG[D] = _make_sc_agg(D)
    out = _SC_AGG[D](h, asrc_flat, adc_flat, rptr, ssrc, m8)
    return out.reshape(N, D)


def _pool_body(h_ref, b3_ref, batch_ref, A1, bA1, A2, bA2, A3, bA3,
               C1, bC1, C2, bC2, C3, bC3, logits_ref, value_ref,
               gacc, cacc):
    k = pl.program_id(0)

    @pl.when(k == 0)
    def _init():
        gacc[...] = jnp.zeros_like(gacc)
        cacc[...] = jnp.zeros_like(cacc)

    act = jnp.maximum(h_ref[...] + b3_ref[...], 0.0)
    oh = jnp.where(
        lax.broadcasted_iota(jnp.int32, (B, act.shape[0]), 0)
        == batch_ref[0], 1.0, 0.0)
    gacc[...] = gacc[...] + jnp.dot(oh, act, preferred_element_type=jnp.float32)
    cacc[...] = cacc[...] + jnp.sum(oh, axis=1, keepdims=True)

    @pl.when(k == pl.num_programs(0) - 1)
    def _final():
        g = gacc[...] / jnp.maximum(cacc[...], 1.0)
        a = jnp.maximum(jnp.dot(g, A1[...], preferred_element_type=jnp.float32)
                        + bA1[...], 0.0)
        a = jnp.maximum(jnp.dot(a, A2[...], preferred_element_type=jnp.float32)
                        + bA2[...], 0.0)
        logits_ref[...] = jnp.tanh(
            jnp.dot(a, A3[...], preferred_element_type=jnp.float32) + bA3[...])
        c = jnp.maximum(jnp.dot(g, C1[...], preferred_element_type=jnp.float32)
                        + bC1[...], 0.0)
        c = jnp.maximum(jnp.dot(c, C2[...], preferred_element_type=jnp.float32)
                        + bC2[...], 0.0)
        value_ref[...] = (jnp.dot(c, C3[...], preferred_element_type=jnp.float32)
                          + bC3[...])


def _pool_heads(h3, b3, batch, p):
    R = 1000
    grid = N // R
    D3 = h3.shape[1]
    full = lambda shape: pl.BlockSpec(shape, lambda i: tuple(0 for _ in shape))
    logits, value = pl.pallas_call(
        _pool_body,
        grid=(grid,),
        in_specs=[
            pl.BlockSpec((R, D3), lambda i: (i, 0)),
            full((1, D3)),
            pl.BlockSpec((1, 1, R), lambda i: (i, 0, 0)),
            full((D3, 512)), full((1, 512)),
            full((512, 1024)), full((1, 1024)),
            full((1024, NUM_SAT * 2)), full((1, NUM_SAT * 2)),
            full((D3, 128)), full((1, 128)),
            full((128, 64)), full((1, 64)),
            full((64, 1)), full((1, 1)),
        ],
        out_specs=[full((B, NUM_SAT * 2)), full((B, 1))],
        out_shape=[
            jax.ShapeDtypeStruct((B, NUM_SAT * 2), jnp.float32),
            jax.ShapeDtypeStruct((B, 1), jnp.float32),
        ],
        scratch_shapes=[
            pltpu.VMEM((B, D3), jnp.float32),
            pltpu.VMEM((B, 1), jnp.float32),
        ],
    )(h3, b3.reshape(1, D3), batch.reshape(grid, 1, R).astype(jnp.int32),
      p['A1'], p['bA1'].reshape(1, 512),
      p['A2'], p['bA2'].reshape(1, 1024),
      p['A3'], p['bA3'].reshape(1, NUM_SAT * 2),
      p['C1'], p['bC1'].reshape(1, 128),
      p['C2'], p['bC2'].reshape(1, 64),
      p['C3'], p['bC3'].reshape(1, 1))
    return logits, value


def kernel(x, params, edge_index, batch):
    p = params
    loop = jnp.arange(N, dtype=jnp.int32)
    src = jnp.concatenate([edge_index[0].astype(jnp.int32), loop])
    dst = jnp.concatenate([edge_index[1].astype(jnp.int32), loop])
    order = jnp.argsort(dst)
    ssrc = src[order]
    sdst = dst[order]
    ssrc = jnp.concatenate(
        [ssrc, jnp.zeros((EP - ssrc.shape[0],), jnp.int32)])
    rptr = jnp.searchsorted(sdst, jnp.arange(N + 1, dtype=jnp.int32)
                            ).astype(jnp.int32)
    rptr = jnp.concatenate(
        [rptr, jnp.full((RP_PAD - N - 1,), E + N, jnp.int32)])

    z = x
    b_prev = jnp.zeros((x.shape[1],), jnp.float32)
    layer_defs = [
        ('W1', 'a1s', 'a1d', 'b1', 256, False),
        ('W2', 'a2s', 'a2d', 'b2', 512, True),
        ('W3', 'a3s', 'a3d', 'b3', 1024, True),
    ]
    for (wk, ask, adk, bk, Dout, do_relu) in layer_defs:
        W, a_s, a_d = p[wk], p[ask], p[adk]
        h, asc, adc = _dense(z, b_prev, W, a_s, a_d, do_relu)
        m8 = jnp.concatenate(
            [jnp.max(asc, axis=0) + jnp.max(adc, axis=0),
             jnp.zeros((124,), jnp.float32)])
        asrc_flat = asc.reshape(N * HEADS)
        adc_flat = jnp.concatenate(
            [adc.reshape(N * HEADS),
             jnp.zeros((ADC_PAD - N * HEADS,), jnp.float32)])
        z = _sc_agg(Dout, h, asrc_flat, adc_flat, rptr, ssrc, m8)
        b_prev = p[bk]

    logits, value = _pool_heads(z, p['b3'], batch, p)
    return (logits.reshape(B, NUM_SAT, 2), value)
